# Initial kernel scaffold; baseline (speedup 1.0000x reference)
import jax
import jax.numpy as jnp
from jax.experimental import pallas as pl


def kernel(x, mask, W1, b1, gn_w, gn_b, W2, b2):
    raise NotImplementedError("write your pallas kernel here")


# speedup vs baseline: 4.6656x; 4.6656x over previous
import functools

import jax
import jax.numpy as jnp
from jax import lax
from jax.experimental import pallas as pl
from jax.experimental.pallas import tpu as pltpu
from jax.experimental.pallas import tpu_sc as plsc

B_, N_, D_, C_ = 8, 2048, 64, 64
K_ = 20
TM = 256
TMC = 256
GSIZE = 4
NC, NS = 2, 16
NW = NC * NS
TOTAL = B_ * N_ * K_
PER_W = TOTAL // NW
CH = 128
QW = 128


NCHUNK = N_ // 128


def _topk_kernel(x_rows, xt_full, w1, b1v, idx_ref, p_ref, q_ref):
    b = pl.program_id(0)
    t = pl.program_id(1)
    xt = x_rows[0]
    xfT = xt_full[0]
    W1 = w1[...]
    W1a = W1[:, :D_]
    W1b = W1[:, D_:]
    dnT = (((1,), (1,)), ((), ()))
    dn0 = (((1,), (0,)), ((), ()))
    p = lax.dot_general(xt, W1a - W1b, dnT,
                        preferred_element_type=jnp.float32) + b1v[...]
    q = lax.dot_general(xt, W1b, dnT, preferred_element_type=jnp.float32)
    p_ref[0] = p
    q_ref[0] = jnp.concatenate([q, jnp.zeros((TM, QW - C_), jnp.float32)], axis=1)

    inf = jnp.float32(jnp.inf)
    row = t * TM + lax.broadcasted_iota(jnp.int32, (TM, 128), 0)
    lane = lax.broadcasted_iota(jnp.int32, (TM, 128), 1)

    chunks = []
    for c in range(NCHUNK):
        xc = xfT[:, c * 128:(c + 1) * 128]
        x2c = jnp.sum(xc * xc, axis=0, keepdims=True)
        xyc = lax.dot_general(xt, xc, dn0, precision=lax.Precision.HIGHEST,
                              preferred_element_type=jnp.float32)
        dc = x2c - 2.0 * xyc
        dc = jnp.where(lane + c * 128 == row, inf, dc)
        chunks.append(dc)
    d3 = jnp.stack(chunks, axis=0)
    col3 = (lax.broadcasted_iota(jnp.int32, (NCHUNK, TM, 128), 2)
            + 128 * lax.broadcasted_iota(jnp.int32, (NCHUNK, TM, 128), 0))

    def _allmin(a):
        for sh in (64, 32, 16, 8, 4, 2, 1):
            a = jnp.minimum(a, pltpu.roll(a, sh, 1))
        return a

    acc0 = jnp.zeros((TM, 128), jnp.int32)

    def body(k, carry):
        d3, acc = carry
        m = _allmin(jnp.min(d3, axis=0))
        key3 = jnp.where(d3 == m[None], col3, N_)
        j = _allmin(jnp.min(key3, axis=0))
        acc = jnp.where(lane == k, j, acc)
        d3 = jnp.where(col3 == j[None], inf, d3)
        return d3, acc

    _, acc = lax.fori_loop(0, K_, body, (d3, acc0))
    idx_ref[0] = acc[:, :K_] + b * N_


def _topk_call(x, W1, b1):
    grid = (B_, N_ // TM)
    return pl.pallas_call(
        _topk_kernel,
        grid=grid,
        in_specs=[
            pl.BlockSpec((1, TM, D_), lambda b, t: (b, t, 0)),
            pl.BlockSpec((1, D_, N_), lambda b, t: (b, 0, 0)),
            pl.BlockSpec((C_, 2 * D_), lambda b, t: (0, 0)),
            pl.BlockSpec((1, C_), lambda b, t: (0, 0)),
        ],
        out_specs=[
            pl.BlockSpec((1, TM, K_), lambda b, t: (b, t, 0)),
            pl.BlockSpec((1, TM, C_), lambda b, t: (b, t, 0)),
            pl.BlockSpec((1, TM, QW), lambda b, t: (b, t, 0)),
        ],
        out_shape=[
            jax.ShapeDtypeStruct((B_, N_, K_), jnp.int32),
            jax.ShapeDtypeStruct((B_, N_, C_), jnp.float32),
            jax.ShapeDtypeStruct((B_, N_, QW), jnp.float32),
        ],
    )(x, jnp.swapaxes(x, 1, 2), W1, b1.reshape(1, C_))


def _gather_call(q_flat, idx_flat):
    mesh = plsc.VectorSubcoreMesh(core_axis_name="c", subcore_axis_name="s")

    @functools.partial(
        pl.kernel,
        mesh=mesh,
        out_type=jax.ShapeDtypeStruct((TOTAL, QW), jnp.float32),
        scratch_types=[
            pltpu.VMEM((CH,), jnp.int32),
            pltpu.VMEM((CH, QW), jnp.float32),
            pltpu.SemaphoreType.DMA,
        ],
    )
    def gk(q_hbm, idx_hbm, out_hbm, idx_v, rows_v, sem):
        wid = lax.axis_index("s") * NC + lax.axis_index("c")
        base = wid * PER_W

        def body(g, carry):
            off = base + g * CH
            pltpu.sync_copy(idx_hbm.at[pl.ds(off, CH)], idx_v)
            pltpu.async_copy(q_hbm.at[idx_v], rows_v, sem).wait()
            pltpu.sync_copy(rows_v, out_hbm.at[pl.ds(off, CH)])
            return carry

        lax.fori_loop(0, PER_W // CH, body, 0)

    return gk(q_flat, idx_flat)


def _mlp_kernel(p_ref, qg_ref, gw, gb, w2, b2v, o_ref):
    p = p_ref[...]
    qg = qg_ref[...][:, :C_]
    h = (qg.reshape(TMC, K_, C_) + p[:, None, :]).reshape(TMC * K_, C_)
    ci = lax.broadcasted_iota(jnp.int32, (C_, C_), 0) // GSIZE
    cj = lax.broadcasted_iota(jnp.int32, (C_, C_), 1) // GSIZE
    G = jnp.where(ci == cj, 1.0 / GSIZE, 0.0).astype(jnp.float32)
    dn0 = (((1,), (0,)), ((), ()))
    m = lax.dot_general(h, G, dn0, preferred_element_type=jnp.float32)
    ms = lax.dot_general(h * h, G, dn0, preferred_element_type=jnp.float32)
    var = ms - m * m
    gn = (h - m) * lax.rsqrt(var + 1e-5) * gw[...] + gb[...]
    r = jnp.maximum(gn, 0.0)
    dnT = (((1,), (1,)), ((), ()))
    o = lax.dot_general(r, w2[...], dnT,
                        preferred_element_type=jnp.float32) + b2v[...]
    o_ref[...] = jnp.max(o.reshape(TMC, K_, C_), axis=1)


def _mlp_call(p_flat, qg, gn_w, gn_b, W2, b2):
    grid = (B_ * N_ // TMC,)
    return pl.pallas_call(
        _mlp_kernel,
        grid=grid,
        in_specs=[
            pl.BlockSpec((TMC, C_), lambda i: (i, 0)),
            pl.BlockSpec((TMC * K_, QW), lambda i: (i, 0)),
            pl.BlockSpec((1, C_), lambda i: (0, 0)),
            pl.BlockSpec((1, C_), lambda i: (0, 0)),
            pl.BlockSpec((C_, C_), lambda i: (0, 0)),
            pl.BlockSpec((1, C_), lambda i: (0, 0)),
        ],
        out_specs=pl.BlockSpec((TMC, C_), lambda i: (i, 0)),
        out_shape=jax.ShapeDtypeStruct((B_ * N_, C_), jnp.float32),
    )(p_flat, qg, gn_w.reshape(1, C_), gn_b.reshape(1, C_), W2,
      b2.reshape(1, C_))


def kernel(x, mask, W1, b1, gn_w, gn_b, W2, b2):
    idx, p, q = _topk_call(x, W1, b1)
    qg = _gather_call(q.reshape(B_ * N_, QW), idx.reshape(TOTAL))
    out = _mlp_call(p.reshape(B_ * N_, C_), qg, gn_w, gn_b, W2, b2)
    out = out.reshape(B_, N_, C_)
    return jnp.where(mask[:, :, None], out, 0.0)


# speedup vs baseline: 5.6599x; 1.2131x over previous
import functools

import jax
import jax.numpy as jnp
from jax import lax
from jax.experimental import pallas as pl
from jax.experimental.pallas import tpu as pltpu
from jax.experimental.pallas import tpu_sc as plsc

B_, N_, D_, C_ = 8, 2048, 64, 64
K_ = 20
TM = 256
TMC = 256
GSIZE = 4
NC, NS = 2, 16
NW = NC * NS
TOTAL = B_ * N_ * K_
PER_W = TOTAL // NW
CH = 128
QW = 128


TR = 128
NPG = N_ // 8
KP = 24


def _topk_kernel(x_rows, x_full, w1, b1v, idx_ref, p_ref, q_ref):
    b = pl.program_id(0)
    t = pl.program_id(1)
    xt = x_rows[0]
    xf = x_full[0]
    W1 = w1[...]
    W1a = W1[:, :D_]
    W1b = W1[:, D_:]
    dnT = (((1,), (1,)), ((), ()))
    dn0 = (((1,), (0,)), ((), ()))
    p = lax.dot_general(xt, W1a - W1b, dnT,
                        preferred_element_type=jnp.float32) + b1v[...]
    q = lax.dot_general(xt, W1b, dnT, preferred_element_type=jnp.float32)
    p_ref[0] = p
    q_ref[0] = jnp.concatenate([q, jnp.zeros((TR, QW - C_), jnp.float32)], axis=1)

    inf = jnp.float32(jnp.inf)
    xtT = lax.transpose(xt, (1, 0))
    xy = lax.dot_general(xf, xtT, dn0, precision=lax.Precision.HIGHEST,
                         preferred_element_type=jnp.float32)
    x2c = lax.dot_general(xf * xf, jnp.ones((D_, TR), jnp.float32), dn0,
                          preferred_element_type=jnp.float32)
    d3 = (x2c - 2.0 * xy).reshape(NPG, 8, TR)
    nbr3 = (8 * lax.broadcasted_iota(jnp.int32, (NPG, 8, TR), 0)
            + lax.broadcasted_iota(jnp.int32, (NPG, 8, TR), 1))
    rowl = t * TR + lax.broadcasted_iota(jnp.int32, (NPG, 8, TR), 2)
    d3 = jnp.where(nbr3 == rowl, inf, d3)

    kio = lax.broadcasted_iota(jnp.int32, (KP, TR), 0)
    acc0 = jnp.zeros((KP, TR), jnp.int32)

    def body(k, carry):
        d3, acc = carry
        m = jnp.min(jnp.min(d3, axis=0), axis=0, keepdims=True)
        key3 = jnp.where(d3 == m, nbr3, N_)
        j = jnp.min(jnp.min(key3, axis=0), axis=0, keepdims=True)
        acc = jnp.where(kio == k, j, acc)
        d3 = jnp.where(nbr3 == j, inf, d3)
        return d3, acc

    _, acc = lax.fori_loop(0, K_, body, (d3, acc0))
    accT = lax.transpose(acc, (1, 0))
    idx_ref[0] = accT[:, :K_] + b * N_


def _topk_call(x, W1, b1):
    grid = (B_, N_ // TR)
    return pl.pallas_call(
        _topk_kernel,
        grid=grid,
        in_specs=[
            pl.BlockSpec((1, TR, D_), lambda b, t: (b, t, 0)),
            pl.BlockSpec((1, N_, D_), lambda b, t: (b, 0, 0)),
            pl.BlockSpec((C_, 2 * D_), lambda b, t: (0, 0)),
            pl.BlockSpec((1, C_), lambda b, t: (0, 0)),
        ],
        out_specs=[
            pl.BlockSpec((1, TR, K_), lambda b, t: (b, t, 0)),
            pl.BlockSpec((1, TR, C_), lambda b, t: (b, t, 0)),
            pl.BlockSpec((1, TR, QW), lambda b, t: (b, t, 0)),
        ],
        out_shape=[
            jax.ShapeDtypeStruct((B_, N_, K_), jnp.int32),
            jax.ShapeDtypeStruct((B_, N_, C_), jnp.float32),
            jax.ShapeDtypeStruct((B_, N_, QW), jnp.float32),
        ],
    )(x, x, W1, b1.reshape(1, C_))


def _gather_call(q_flat, idx_flat):
    mesh = plsc.VectorSubcoreMesh(core_axis_name="c", subcore_axis_name="s")

    @functools.partial(
        pl.kernel,
        mesh=mesh,
        out_type=jax.ShapeDtypeStruct((TOTAL, QW), jnp.float32),
        scratch_types=[
            pltpu.VMEM((CH,), jnp.int32),
            pltpu.VMEM((CH, QW), jnp.float32),
            pltpu.SemaphoreType.DMA,
        ],
    )
    def gk(q_hbm, idx_hbm, out_hbm, idx_v, rows_v, sem):
        wid = lax.axis_index("s") * NC + lax.axis_index("c")
        base = wid * PER_W

        def body(g, carry):
            off = base + g * CH
            pltpu.sync_copy(idx_hbm.at[pl.ds(off, CH)], idx_v)
            pltpu.async_copy(q_hbm.at[idx_v], rows_v, sem).wait()
            pltpu.sync_copy(rows_v, out_hbm.at[pl.ds(off, CH)])
            return carry

        lax.fori_loop(0, PER_W // CH, body, 0)

    return gk(q_flat, idx_flat)


def _mlp_kernel(p_ref, qg_ref, gw, gb, w2, b2v, o_ref):
    p = p_ref[...]
    qg = qg_ref[...][:, :C_]
    h = (qg.reshape(TMC, K_, C_) + p[:, None, :]).reshape(TMC * K_, C_)
    ci = lax.broadcasted_iota(jnp.int32, (C_, C_), 0) // GSIZE
    cj = lax.broadcasted_iota(jnp.int32, (C_, C_), 1) // GSIZE
    G = jnp.where(ci == cj, 1.0 / GSIZE, 0.0).astype(jnp.float32)
    dn0 = (((1,), (0,)), ((), ()))
    m = lax.dot_general(h, G, dn0, preferred_element_type=jnp.float32)
    ms = lax.dot_general(h * h, G, dn0, preferred_element_type=jnp.float32)
    var = ms - m * m
    gn = (h - m) * lax.rsqrt(var + 1e-5) * gw[...] + gb[...]
    r = jnp.maximum(gn, 0.0)
    dnT = (((1,), (1,)), ((), ()))
    o = lax.dot_general(r, w2[...], dnT,
                        preferred_element_type=jnp.float32) + b2v[...]
    o_ref[...] = jnp.max(o.reshape(TMC, K_, C_), axis=1)


def _mlp_call(p_flat, qg, gn_w, gn_b, W2, b2):
    grid = (B_ * N_ // TMC,)
    return pl.pallas_call(
        _mlp_kernel,
        grid=grid,
        in_specs=[
            pl.BlockSpec((TMC, C_), lambda i: (i, 0)),
            pl.BlockSpec((TMC * K_, QW), lambda i: (i, 0)),
            pl.BlockSpec((1, C_), lambda i: (0, 0)),
            pl.BlockSpec((1, C_), lambda i: (0, 0)),
            pl.BlockSpec((C_, C_), lambda i: (0, 0)),
            pl.BlockSpec((1, C_), lambda i: (0, 0)),
        ],
        out_specs=pl.BlockSpec((TMC, C_), lambda i: (i, 0)),
        out_shape=jax.ShapeDtypeStruct((B_ * N_, C_), jnp.float32),
    )(p_flat, qg, gn_w.reshape(1, C_), gn_b.reshape(1, C_), W2,
      b2.reshape(1, C_))


def kernel(x, mask, W1, b1, gn_w, gn_b, W2, b2):
    idx, p, q = _topk_call(x, W1, b1)
    qg = _gather_call(q.reshape(B_ * N_, QW), idx.reshape(TOTAL))
    out = _mlp_call(p.reshape(B_ * N_, C_), qg, gn_w, gn_b, W2, b2)
    out = out.reshape(B_, N_, C_)
    return jnp.where(mask[:, :, None], out, 0.0)
